# baseline (speedup 1.0000x reference)
import functools

import jax
import jax.numpy as jnp
from jax import lax
from jax.experimental import pallas as pl
from jax.experimental.pallas import tpu as pltpu
from jax.experimental.pallas import tpu_sc as plsc

N = 10000
E = 320000
DIN = 128
DH = 256
DOUT = 128
G = 64

NUM_TILES = 16
CHUNK = 128
NPAD = 10240
RPT = NPAD // NUM_TILES
EPAD = -(-E // (64 * CHUNK)) * (64 * CHUNK)
EPT1 = EPAD // 32
NCHUNK1 = EPT1 // CHUNK
EPT2 = EPAD // NUM_TILES
NCHUNK2 = EPT2 // CHUNK
BM = 512
NBLK = NPAD // BM

_MESH = plsc.VectorSubcoreMesh(core_axis_name="c", subcore_axis_name="s")


def _edge_loop(table, srcp, dst, acc, bufs, ebase_src, ebase_dst, nchunk):
    is_, id_, r, gsem, ssem = bufs

    def load_idx(j, b):
        pltpu.sync_copy(srcp.at[pl.ds(ebase_src + j * CHUNK, CHUNK)], is_[b])
        pltpu.sync_copy(dst.at[pl.ds(ebase_dst + j * CHUNK, CHUNK)], id_[b])

    def gather(b):
        pltpu.async_copy(table.at[is_[b]], r[b], gsem[b])

    def gwait(b):
        pltpu.make_async_copy(table.at[is_[b]], r[b], gsem[b]).wait()

    def scatter(b):
        pltpu.async_copy(r[b], acc.at[id_[b]], ssem[b], add=True)

    def swait(b):
        pltpu.make_async_copy(r[b], acc.at[id_[b]], ssem[b]).wait()

    load_idx(0, 0)
    gather(0)
    load_idx(1, 1)
    gather(1)
    gwait(0)
    scatter(0)

    def body(i2, carry):
        for half in (0, 1):
            j = 1 + i2 * 2 + half
            b = (1 + half) % 2
            nb = 1 - b
            swait(nb)
            load_idx(j + 1, nb)
            gather(nb)
            gwait(b)
            scatter(b)
        return carry

    lax.fori_loop(0, (nchunk - 2) // 2, body, 0)
    bl = (nchunk - 1) % 2
    swait(1 - bl)
    gwait(bl)
    scatter(bl)
    swait(bl)


def _sc_scratch(dc):
    return [
        [pltpu.VMEM((CHUNK,), jnp.int32)] * 2,
        [pltpu.VMEM((CHUNK,), jnp.int32)] * 2,
        [pltpu.VMEM((CHUNK, dc), jnp.float32)] * 2,
        [pltpu.SemaphoreType.DMA] * 2,
        [pltpu.SemaphoreType.DMA] * 2,
        pltpu.VMEM_SHARED((NPAD, dc), jnp.float32),
    ]


@functools.partial(
    pl.kernel,
    out_type=jax.ShapeDtypeStruct((2, NPAD, DIN), jnp.float32),
    mesh=_MESH,
    scratch_types=_sc_scratch(DIN),
)
def _sc_agg1(table, src1d, dst1d, zeros, z,
             is_, id_, r, gsem, ssem, acc):
    c = lax.axis_index("c")
    s = lax.axis_index("s")
    rr = s * RPT

    @pl.when(c == 0)
    def _():
        pltpu.sync_copy(table.at[pl.ds(rr, RPT)], acc.at[pl.ds(rr, RPT)])

    @pl.when(c == 1)
    def _():
        pltpu.sync_copy(zeros, acc.at[pl.ds(rr, RPT)])

    plsc.subcore_barrier()
    ebase = (c * NUM_TILES + s) * EPT1
    _edge_loop(table, src1d, dst1d, acc,
               (is_, id_, r, gsem, ssem),
               ebase, ebase, NCHUNK1)
    plsc.subcore_barrier()
    pltpu.sync_copy(acc.at[pl.ds(rr, RPT)], z.at[c, pl.ds(rr, RPT)])


@functools.partial(
    pl.kernel,
    out_type=jax.ShapeDtypeStruct((2, NPAD, DH // 2), jnp.float32),
    mesh=_MESH,
    scratch_types=_sc_scratch(DH // 2),
)
def _sc_agg2(table, srcp1d, dst1d, z,
             is_, id_, r, gsem, ssem, acc):
    c = lax.axis_index("c")
    s = lax.axis_index("s")
    rr = s * RPT
    pltpu.sync_copy(table.at[pl.ds(c * NPAD + rr, RPT)],
                    acc.at[pl.ds(rr, RPT)])
    plsc.subcore_barrier()
    _edge_loop(table, srcp1d, dst1d, acc,
               (is_, id_, r, gsem, ssem),
               c * EPAD + s * EPT2, s * EPT2, NCHUNK2)
    plsc.subcore_barrier()
    pltpu.sync_copy(acc.at[pl.ds(rr, RPT)], z.at[c, pl.ds(rr, RPT)])


def _mm1_body(z_ref, w_ref, b_ref, out_ref):
    zsum = z_ref[0] + z_ref[1]
    h = jnp.dot(zsum, w_ref[...], preferred_element_type=jnp.float32)
    h = jnp.maximum(h + b_ref[...], 0.0)
    out_ref[0] = h[:, : DH // 2]
    out_ref[1] = h[:, DH // 2:]


def _tc_mm1(z, w, b):
    return pl.pallas_call(
        _mm1_body,
        grid=(NBLK,),
        in_specs=[
            pl.BlockSpec((2, BM, DIN), lambda i: (0, i, 0)),
            pl.BlockSpec((DIN, DH), lambda i: (0, 0)),
            pl.BlockSpec((1, DH), lambda i: (0, 0)),
        ],
        out_specs=pl.BlockSpec((2, BM, DH // 2), lambda i: (0, i, 0)),
        out_shape=jax.ShapeDtypeStruct((2, NPAD, DH // 2), jnp.float32),
    )(z, w, b)


def _mm2_body(z_ref, w2_ref, b2_ref, batch_ref, w3_ref, b3_ref, out_ref,
              acc_ref, cnt_ref):
    i = pl.program_id(0)

    @pl.when(i == 0)
    def _():
        acc_ref[...] = jnp.zeros_like(acc_ref)
        cnt_ref[...] = jnp.zeros_like(cnt_ref)

    h = jnp.dot(z_ref[0], w2_ref[...][: DH // 2],
                preferred_element_type=jnp.float32)
    h += jnp.dot(z_ref[1], w2_ref[...][DH // 2:],
                 preferred_element_type=jnp.float32)
    h = jnp.maximum(h + b2_ref[...], 0.0)
    gid = lax.broadcasted_iota(jnp.int32, (BM, G), 1)
    onehot = (batch_ref[...] == gid).astype(jnp.float32)
    acc_ref[...] += lax.dot_general(
        onehot, h, (((0,), (0,)), ((), ())),
        preferred_element_type=jnp.float32)
    cnt_ref[...] += lax.dot_general(
        onehot, jnp.ones((BM, 1), jnp.float32), (((0,), (0,)), ((), ())),
        preferred_element_type=jnp.float32)

    @pl.when(i == NBLK - 1)
    def _():
        pooled = acc_ref[...] / jnp.maximum(cnt_ref[...], 1.0)
        out_ref[...] = jnp.dot(
            pooled, w3_ref[...], preferred_element_type=jnp.float32
        ) + b3_ref[...]


def _tc_mm2(z, w2, b2, batch2d, w3, b3):
    return pl.pallas_call(
        _mm2_body,
        grid=(NBLK,),
        in_specs=[
            pl.BlockSpec((2, BM, DH // 2), lambda i: (0, i, 0)),
            pl.BlockSpec((DH, DH), lambda i: (0, 0)),
            pl.BlockSpec((1, DH), lambda i: (0, 0)),
            pl.BlockSpec((BM, 1), lambda i: (i, 0)),
            pl.BlockSpec((DH, DOUT), lambda i: (0, 0)),
            pl.BlockSpec((1, DOUT), lambda i: (0, 0)),
        ],
        out_specs=pl.BlockSpec((G, DOUT), lambda i: (0, 0)),
        out_shape=jax.ShapeDtypeStruct((G, DOUT), jnp.float32),
        scratch_shapes=[
            pltpu.VMEM((G, DH), jnp.float32),
            pltpu.VMEM((G, 1), jnp.float32),
        ],
    )(z, w2, b2, batch2d, w3, b3)


def kernel(x, edge_index, batch, W1, b1, W2, b2, W3, b3):
    src = edge_index[0]
    dst = edge_index[1]
    pad_e = EPAD - E
    src_p = jnp.concatenate([src, jnp.zeros((pad_e,), jnp.int32)])
    dst_p = jnp.concatenate([dst, jnp.full((pad_e,), N, jnp.int32)])
    srcp2 = jnp.concatenate([src_p, src_p + NPAD])

    xpad = jnp.pad(x, ((0, NPAD - N), (0, 0)))
    zeros_tile = jnp.zeros((RPT, DIN), jnp.float32)

    z1 = _sc_agg1(xpad, src_p, dst_p, zeros_tile)
    h1 = _tc_mm1(z1, W1, b1.reshape(1, DH))
    z2 = _sc_agg2(h1.reshape(2 * NPAD, DH // 2), srcp2, dst_p)

    batch_p = jnp.pad(batch, (0, NPAD - N), constant_values=G)
    return _tc_mm2(z2, W2, b2.reshape(1, DH), batch_p.reshape(NPAD, 1),
                   W3, b3.reshape(1, DOUT))


# speedup vs baseline: 1.0864x; 1.0864x over previous
import functools

import jax
import jax.numpy as jnp
from jax import lax
from jax.experimental import pallas as pl
from jax.experimental.pallas import tpu as pltpu
from jax.experimental.pallas import tpu_sc as plsc

N = 10000
E = 320000
DIN = 128
DH = 256
DOUT = 128
G = 64

NUM_TILES = 16
CHUNK = 128
NPAD = 10240
RPT = NPAD // NUM_TILES
EPAD = -(-E // (64 * CHUNK)) * (64 * CHUNK)
EPT1 = EPAD // 32
NCHUNK1 = EPT1 // CHUNK
EPT2 = EPAD // NUM_TILES
NCHUNK2 = EPT2 // CHUNK
BM = 512
NBLK = NPAD // BM

_MESH = plsc.VectorSubcoreMesh(core_axis_name="c", subcore_axis_name="s")


BLKC = 8
BLKE = BLKC * CHUNK


def _edge_loop(table, srcp, dst, acc, bufs, ebase_src, ebase_dst, nchunk):
    sbuf, dbuf, r, gsem, ssem = bufs

    def load_blk(k, p):
        pltpu.sync_copy(srcp.at[pl.ds(ebase_src + k * BLKE, BLKE)], sbuf[p])
        pltpu.sync_copy(dst.at[pl.ds(ebase_dst + k * BLKE, BLKE)], dbuf[p])

    def sidx(m):
        return sbuf[(m // BLKC) % 2].at[pl.ds((m % BLKC) * CHUNK, CHUNK)]

    def didx(m):
        return dbuf[(m // BLKC) % 2].at[pl.ds((m % BLKC) * CHUNK, CHUNK)]

    def gather(m):
        pltpu.async_copy(table.at[sidx(m)], r[m % 2], gsem[m % 2])

    def gwait(m):
        pltpu.make_async_copy(table.at[sidx(m)], r[m % 2],
                              gsem[m % 2]).wait()

    def scatter(m):
        pltpu.async_copy(r[m % 2], acc.at[didx(m)], ssem[m % 2], add=True)

    def swait(m):
        pltpu.make_async_copy(r[m % 2], acc.at[didx(m)], ssem[m % 2]).wait()

    def step(j, m):
        if m != 0:
            swait(m - 1)
        else:
            swait(15)
        if m % BLKC == 0:
            load_blk(j // BLKC + 1, (m // BLKC + 1) % 2)
        gather(m + 1)
        gwait(m)
        scatter(m)

    load_blk(0, 0)
    gather(0)
    load_blk(1, 1)
    gather(1)
    gwait(0)
    scatter(0)
    for j in range(1, 16):
        step(j, j)

    def body(k2, carry):
        j0 = 16 + k2 * 16
        for m in range(16):
            step(j0 + m, m)
        return carry

    lax.fori_loop(0, (nchunk - 16) // 16, body, 0)
    swait(nchunk - 1)
    gwait(nchunk)


def _sc_scratch(dc):
    return [
        [pltpu.VMEM((BLKE,), jnp.int32)] * 2,
        [pltpu.VMEM((BLKE,), jnp.int32)] * 2,
        [pltpu.VMEM((CHUNK, dc), jnp.float32)] * 2,
        [pltpu.SemaphoreType.DMA] * 2,
        [pltpu.SemaphoreType.DMA] * 2,
        pltpu.VMEM_SHARED((NPAD, dc), jnp.float32),
    ]


@functools.partial(
    pl.kernel,
    out_type=jax.ShapeDtypeStruct((2, NPAD, DIN), jnp.float32),
    mesh=_MESH,
    scratch_types=_sc_scratch(DIN),
)
def _sc_agg1(table, src1d, dst1d, zeros, z,
             sbuf, dbuf, r, gsem, ssem, acc):
    c = lax.axis_index("c")
    s = lax.axis_index("s")
    rr = s * RPT

    @pl.when(c == 0)
    def _():
        pltpu.sync_copy(table.at[pl.ds(rr, RPT)], acc.at[pl.ds(rr, RPT)])

    @pl.when(c == 1)
    def _():
        pltpu.sync_copy(zeros, acc.at[pl.ds(rr, RPT)])

    plsc.subcore_barrier()
    ebase = (c * NUM_TILES + s) * EPT1
    _edge_loop(table, src1d, dst1d, acc,
               (sbuf, dbuf, r, gsem, ssem),
               ebase, ebase, NCHUNK1)
    plsc.subcore_barrier()
    pltpu.sync_copy(acc.at[pl.ds(rr, RPT)], z.at[c, pl.ds(rr, RPT)])


@functools.partial(
    pl.kernel,
    out_type=jax.ShapeDtypeStruct((2, NPAD, DH // 2), jnp.float32),
    mesh=_MESH,
    scratch_types=_sc_scratch(DH // 2),
)
def _sc_agg2(table, srcp1d, dst1d, z,
             sbuf, dbuf, r, gsem, ssem, acc):
    c = lax.axis_index("c")
    s = lax.axis_index("s")
    rr = s * RPT
    pltpu.sync_copy(table.at[pl.ds(c * NPAD + rr, RPT)],
                    acc.at[pl.ds(rr, RPT)])
    plsc.subcore_barrier()
    _edge_loop(table, srcp1d, dst1d, acc,
               (sbuf, dbuf, r, gsem, ssem),
               c * EPAD + s * EPT2, s * EPT2, NCHUNK2)
    plsc.subcore_barrier()
    pltpu.sync_copy(acc.at[pl.ds(rr, RPT)], z.at[c, pl.ds(rr, RPT)])


def _mm1_body(z_ref, w_ref, b_ref, out_ref):
    zsum = z_ref[0] + z_ref[1]
    h = jnp.dot(zsum, w_ref[...], preferred_element_type=jnp.float32)
    h = jnp.maximum(h + b_ref[...], 0.0)
    out_ref[0] = h[:, : DH // 2]
    out_ref[1] = h[:, DH // 2:]


def _tc_mm1(z, w, b):
    return pl.pallas_call(
        _mm1_body,
        grid=(NBLK,),
        in_specs=[
            pl.BlockSpec((2, BM, DIN), lambda i: (0, i, 0)),
            pl.BlockSpec((DIN, DH), lambda i: (0, 0)),
            pl.BlockSpec((1, DH), lambda i: (0, 0)),
        ],
        out_specs=pl.BlockSpec((2, BM, DH // 2), lambda i: (0, i, 0)),
        out_shape=jax.ShapeDtypeStruct((2, NPAD, DH // 2), jnp.float32),
    )(z, w, b)


def _mm2_body(z_ref, w2_ref, b2_ref, batch_ref, w3_ref, b3_ref, out_ref,
              acc_ref, cnt_ref):
    i = pl.program_id(0)

    @pl.when(i == 0)
    def _():
        acc_ref[...] = jnp.zeros_like(acc_ref)
        cnt_ref[...] = jnp.zeros_like(cnt_ref)

    h = jnp.dot(z_ref[0], w2_ref[...][: DH // 2],
                preferred_element_type=jnp.float32)
    h += jnp.dot(z_ref[1], w2_ref[...][DH // 2:],
                 preferred_element_type=jnp.float32)
    h = jnp.maximum(h + b2_ref[...], 0.0)
    gid = lax.broadcasted_iota(jnp.int32, (BM, G), 1)
    onehot = (batch_ref[...] == gid).astype(jnp.float32)
    acc_ref[...] += lax.dot_general(
        onehot, h, (((0,), (0,)), ((), ())),
        preferred_element_type=jnp.float32)
    cnt_ref[...] += lax.dot_general(
        onehot, jnp.ones((BM, 1), jnp.float32), (((0,), (0,)), ((), ())),
        preferred_element_type=jnp.float32)

    @pl.when(i == NBLK - 1)
    def _():
        pooled = acc_ref[...] / jnp.maximum(cnt_ref[...], 1.0)
        out_ref[...] = jnp.dot(
            pooled, w3_ref[...], preferred_element_type=jnp.float32
        ) + b3_ref[...]


def _tc_mm2(z, w2, b2, batch2d, w3, b3):
    return pl.pallas_call(
        _mm2_body,
        grid=(NBLK,),
        in_specs=[
            pl.BlockSpec((2, BM, DH // 2), lambda i: (0, i, 0)),
            pl.BlockSpec((DH, DH), lambda i: (0, 0)),
            pl.BlockSpec((1, DH), lambda i: (0, 0)),
            pl.BlockSpec((BM, 1), lambda i: (i, 0)),
            pl.BlockSpec((DH, DOUT), lambda i: (0, 0)),
            pl.BlockSpec((1, DOUT), lambda i: (0, 0)),
        ],
        out_specs=pl.BlockSpec((G, DOUT), lambda i: (0, 0)),
        out_shape=jax.ShapeDtypeStruct((G, DOUT), jnp.float32),
        scratch_shapes=[
            pltpu.VMEM((G, DH), jnp.float32),
            pltpu.VMEM((G, 1), jnp.float32),
        ],
    )(z, w2, b2, batch2d, w3, b3)


def kernel(x, edge_index, batch, W1, b1, W2, b2, W3, b3):
    src = edge_index[0]
    dst = edge_index[1]
    pad_e = EPAD - E
    src_p = jnp.concatenate([src, jnp.zeros((pad_e,), jnp.int32)])
    dst_p = jnp.concatenate([dst, jnp.full((pad_e,), N, jnp.int32)])
    srcp2 = jnp.concatenate([src_p, src_p + NPAD])
    tail = jnp.zeros((BLKE,), jnp.int32)
    src_p = jnp.concatenate([src_p, tail])
    dst_p = jnp.concatenate([dst_p, tail])
    srcp2 = jnp.concatenate([srcp2, tail])

    xpad = jnp.pad(x, ((0, NPAD - N), (0, 0)))
    zeros_tile = jnp.zeros((RPT, DIN), jnp.float32)

    z1 = _sc_agg1(xpad, src_p, dst_p, zeros_tile)
    h1 = _tc_mm1(z1, W1, b1.reshape(1, DH))
    z2 = _sc_agg2(h1.reshape(2 * NPAD, DH // 2), srcp2, dst_p)

    batch_p = jnp.pad(batch, (0, NPAD - N), constant_values=G)
    return _tc_mm2(z2, W2, b2.reshape(1, DH), batch_p.reshape(NPAD, 1),
                   W3, b3.reshape(1, DOUT))
